# baseline (speedup 1.0000x reference)
import jax
import jax.numpy as jnp
from jax import lax
from jax.experimental import pallas as pl
from jax.experimental.pallas import tpu as pltpu
from jax.experimental.pallas import tpu_sc as plsc

B = 4096
S = 50
D = 64
L = 16
NC = 2
NS = 16
NW = NC * NS
B_PER_W = B // NW
CHUNK_B = 2
IDX_PER_CHUNK = CHUNK_B * S
N_CHUNKS = B_PER_W // CHUNK_B


def kernel(indices, table):
    idx3 = indices.astype(jnp.int32).reshape(NW, N_CHUNKS, IDX_PER_CHUNK)
    mesh = plsc.VectorSubcoreMesh(core_axis_name="c", subcore_axis_name="s")

    @pl.kernel(
        out_type=jax.ShapeDtypeStruct((B, D), jnp.float32),
        mesh=mesh,
        scratch_types=[
            pltpu.VMEM((N_CHUNKS, IDX_PER_CHUNK), jnp.int32),
            pltpu.VMEM((IDX_PER_CHUNK, D), jnp.float32),
            pltpu.VMEM((B_PER_W, D), jnp.float32),
        ],
        compiler_params=pltpu.CompilerParams(use_tc_tiling_on_sc=False),
    )
    def sc_kernel(table_hbm, idx_hbm, out_hbm, idx_v, rows_v, out_v):
        wid = lax.axis_index("s") * NC + lax.axis_index("c")
        pltpu.sync_copy(idx_hbm.at[wid], idx_v)

        @pl.loop(0, N_CHUNKS)
        def _(ci):
            pltpu.sync_copy(table_hbm.at[idx_v.at[ci]], rows_v)

            @pl.loop(0, CHUNK_B)
            def _(b):
                base = b * S
                for d in range(D // L):
                    sl = pl.ds(d * L, L)
                    acc = rows_v[base, sl]
                    for r in range(1, S):
                        acc = acc + rows_v[base + r, sl]
                    out_v[ci * CHUNK_B + b, sl] = acc * (1.0 / S)

        pltpu.sync_copy(out_v, out_hbm.at[pl.ds(wid * B_PER_W, B_PER_W)])

    return sc_kernel(table, idx3)


# speedup vs baseline: 1.1813x; 1.1813x over previous
import jax
import jax.numpy as jnp
from jax import lax
from jax.experimental import pallas as pl
from jax.experimental.pallas import tpu as pltpu
from jax.experimental.pallas import tpu_sc as plsc

B = 4096
S = 50
H = S // 2
D = 64
L = 16
NC = 2
NS = 16
NW = NC * NS
B_PER_W = B // NW
SP = 64


def kernel(indices, table):
    idx = indices.astype(jnp.int32)
    idx3 = jnp.pad(idx, ((0, 0), (0, SP - S))).reshape(NW, B_PER_W, SP)
    table3 = table.reshape(table.shape[0] // 8, 8, D)
    mesh = plsc.VectorSubcoreMesh(core_axis_name="c", subcore_axis_name="s")

    @pl.kernel(
        out_type=jax.ShapeDtypeStruct((B, D), jnp.float32),
        mesh=mesh,
        scratch_types=[
            pltpu.VMEM((B_PER_W, SP), jnp.int32),
            pltpu.VMEM((H, 8, D), jnp.float32),
            pltpu.VMEM((H, 8, D), jnp.float32),
            pltpu.VMEM((B_PER_W, D), jnp.float32),
            pltpu.SemaphoreType.DMA,
            pltpu.SemaphoreType.DMA,
        ],
        compiler_params=pltpu.CompilerParams(use_tc_tiling_on_sc=True),
    )
    def sc_kernel(table_hbm, idx_hbm, out_hbm, idx_v,
                  buf_a, buf_b, out_v, sem_a, sem_b):
        wid = lax.axis_index("s") * NC + lax.axis_index("c")
        pltpu.sync_copy(idx_hbm.at[wid], idx_v)

        def start(b, phase, buf, sem):
            qvecs = [idx_v[b, pl.ds(k * L, L)] >> 3 for k in range(4)]
            for j in range(H):
                t = phase * H + j
                q = qvecs[t // L][t % L]
                pltpu.async_copy(table_hbm.at[q], buf.at[j], sem)

        def wait(buf, sem):
            pltpu.make_async_copy(table_hbm.at[pl.ds(0, H)], buf, sem).wait()

        def accumulate(buf, b, phase):
            rvecs = [idx_v[b, pl.ds(k * L, L)] & 7 for k in range(4)]
            accs = [None] * (D // L)
            for j in range(H):
                t = phase * H + j
                rr = rvecs[t // L][t % L]
                for d in range(D // L):
                    sl = pl.ds(d * L, L)
                    v = buf[j, rr, sl]
                    accs[d] = v if accs[d] is None else accs[d] + v
            for d in range(D // L):
                sl = pl.ds(d * L, L)
                if phase == 0:
                    out_v[b, sl] = accs[d]
                else:
                    out_v[b, sl] = (out_v[b, sl] + accs[d]) * (1.0 / S)

        start(0, 0, buf_a, sem_a)
        start(0, 1, buf_b, sem_b)

        @pl.loop(0, B_PER_W)
        def _(b):
            wait(buf_a, sem_a)
            accumulate(buf_a, b, 0)
            @pl.when(b < B_PER_W - 1)
            def _():
                start(b + 1, 0, buf_a, sem_a)
            wait(buf_b, sem_b)
            accumulate(buf_b, b, 1)
            @pl.when(b < B_PER_W - 1)
            def _():
                start(b + 1, 1, buf_b, sem_b)

        pltpu.sync_copy(out_v, out_hbm.at[pl.ds(wid * B_PER_W, B_PER_W)])

    return sc_kernel(table3, idx3)
